# baseline (speedup 1.0000x reference)
import functools

import jax
import jax.numpy as jnp
from jax import lax
from jax.experimental import pallas as pl
from jax.experimental.pallas import tpu as pltpu
from jax.experimental.pallas import tpu_sc as plsc

D = 128
LANES = 16
NS = 16
K = 128
CPT = 160


def _make_agg(n_users, n_items):
    edges_per_tile = CPT * K
    rows_main = 624
    rows_last = n_users - (NS - 1) * rows_main

    mesh = plsc.VectorSubcoreMesh(core_axis_name="c", subcore_axis_name="s",
                                  num_cores=2, num_subcores=NS)

    @functools.partial(
        pl.kernel,
        out_type=(
            jax.ShapeDtypeStruct((n_users, D), jnp.float32),
            jax.ShapeDtypeStruct((n_items, D), jnp.float32),
        ),
        mesh=mesh,
        scratch_types=[
            pltpu.VMEM_SHARED((n_users, D), jnp.float32),
            pltpu.VMEM((K,), jnp.int32),
            pltpu.VMEM((K,), jnp.int32),
            pltpu.VMEM((K,), jnp.int32),
            pltpu.VMEM((K,), jnp.int32),
            pltpu.VMEM((K,), jnp.float32),
            pltpu.VMEM((K,), jnp.float32),
            pltpu.VMEM((K, D), jnp.float32),
            pltpu.VMEM((K, D), jnp.float32),
            pltpu.SemaphoreType.DMA,
            pltpu.SemaphoreType.DMA,
        ],
    )
    def agg(user_emb, item_emb, row_idx, col_idx, vals,
            out_u, out_i, acc, idxs0, idxs1, idxd0, idxd1, vals0, vals1,
            rows0, rows1, sem0, sem1):
        cid = lax.axis_index("c")
        sid = lax.axis_index("s")
        idxs_b = (idxs0, idxs1)
        idxd_b = (idxd0, idxd1)
        vals_b = (vals0, vals1)
        rows_b = (rows0, rows1)
        sem_b = (sem0, sem1)

        zeros16 = jnp.zeros((LANES,), jnp.float32)

        def zero_row(r, carry):
            for j in range(D // LANES):
                rows0[r, pl.ds(j * LANES, LANES)] = zeros16
            return carry

        lax.fori_loop(0, K, zero_row, 0)

        @pl.when(sid < NS - 1)
        def _():
            base_r = sid * rows_main
            for i in range(6):
                pltpu.sync_copy(rows0.at[pl.ds(0, 104)],
                                acc.at[pl.ds(base_r + i * 104, 104)])

        @pl.when(sid == NS - 1)
        def _():
            base_r = (NS - 1) * rows_main
            for i in range(rows_last // K):
                pltpu.sync_copy(rows0.at[pl.ds(0, K)],
                                acc.at[pl.ds(base_r + i * K, K)])

        plsc.subcore_barrier()

        def run(dst_hbm, src_hbm, emb_hbm, out_hbm):
            def load_chunk(c, b):
                base = sid * edges_per_tile + c * K
                pltpu.sync_copy(src_hbm.at[pl.ds(base, K)], idxs_b[b])
                pltpu.sync_copy(dst_hbm.at[pl.ds(base, K)], idxd_b[b])
                pltpu.sync_copy(vals.at[pl.ds(base, K)], vals_b[b])
                pltpu.async_copy(emb_hbm.at[idxs_b[b]], rows_b[b], sem_b[b])

            def wait_gather(b):
                pltpu.make_async_copy(emb_hbm.at[idxs_b[b]], rows_b[b],
                                      sem_b[b]).wait()

            def scale_scatter(b):
                rows = rows_b[b]
                vls = vals_b[b]

                def scale_grp(g, c2):
                    vals16 = vls[pl.ds(g * LANES, LANES)]
                    for e in range(LANES):
                        v = vals16[e]
                        r = g * LANES + e
                        for j in range(D // LANES):
                            sl = pl.ds(j * LANES, LANES)
                            rows[r, sl] = rows[r, sl] * v
                    return c2

                lax.fori_loop(0, K // LANES, scale_grp, 0)
                pltpu.sync_copy(rows, acc.at[idxd_b[b]], add=True)

            load_chunk(0, 0)

            def pair_body(p, carry):
                c0 = 2 * p
                load_chunk(c0 + 1, 1)
                wait_gather(0)
                scale_scatter(0)

                @pl.when(p < CPT // 2 - 1)
                def _():
                    load_chunk(c0 + 2, 0)

                wait_gather(1)
                scale_scatter(1)
                return carry

            lax.fori_loop(0, CPT // 2, pair_body, 0)
            plsc.subcore_barrier()

            @pl.when(sid < NS - 1)
            def _():
                rb = sid * rows_main
                pltpu.sync_copy(acc.at[pl.ds(rb, rows_main)],
                                out_hbm.at[pl.ds(rb, rows_main)])

            @pl.when(sid == NS - 1)
            def _():
                rb = (NS - 1) * rows_main
                pltpu.sync_copy(acc.at[pl.ds(rb, rows_last)],
                                out_hbm.at[pl.ds(rb, rows_last)])

        @pl.when(cid == 0)
        def _():
            run(row_idx, col_idx, item_emb, out_u)

        @pl.when(cid == 1)
        def _():
            run(col_idx, row_idx, user_emb, out_i)

    return agg


def kernel(user_emb, item_emb, mat_indices, mat_values):
    n_users = user_emb.shape[0]
    n_items = item_emb.shape[0]
    e = mat_values.shape[0]
    e_pad = NS * CPT * K
    pad = e_pad - e
    row = mat_indices[0]
    col = mat_indices[1]
    if pad:
        zi = jnp.zeros((pad,), jnp.int32)
        row = jnp.concatenate([row, zi])
        col = jnp.concatenate([col, zi])
        mat_values = jnp.concatenate([mat_values,
                                      jnp.zeros((pad,), jnp.float32)])
    agg = _make_agg(n_users, n_items)
    return agg(user_emb, item_emb, row, col, mat_values)
